# baseline (speedup 1.0000x reference)
import functools

import jax
import jax.numpy as jnp
from jax import lax
from jax.experimental import pallas as pl
from jax.experimental.pallas import tpu as pltpu
from jax.experimental.pallas import tpu_sc as plsc

D_MODEL = 2048
E = 16
K = 2
N = 8192
NW = 32
TPW = N // NW
CHUNK = 16


def _logits_body(x_ref, w_ref, b_ref, out_ref):
    out_ref[...] = (
        jnp.dot(x_ref[...], w_ref[...], preferred_element_type=jnp.float32)
        + b_ref[...]
    )


def _logits(x, W, b2):
    m_blk = 1024
    return pl.pallas_call(
        _logits_body,
        grid=(N // m_blk,),
        in_specs=[
            pl.BlockSpec((m_blk, D_MODEL), lambda i: (i, 0)),
            pl.BlockSpec((D_MODEL, E), lambda i: (0, 0)),
            pl.BlockSpec((1, E), lambda i: (0, 0)),
        ],
        out_specs=pl.BlockSpec((m_blk, E), lambda i: (i, 0)),
        out_shape=jax.ShapeDtypeStruct((N, E), jnp.float32),
    )(x, W, b2)


_MESH = plsc.VectorSubcoreMesh(core_axis_name="c", subcore_axis_name="s")


@functools.partial(
    pl.kernel,
    out_type=[
        jax.ShapeDtypeStruct((N * K,), jnp.int32),
        jax.ShapeDtypeStruct((N * K,), jnp.float32),
        jax.ShapeDtypeStruct((NW, E), jnp.float32),
    ],
    mesh=_MESH,
    compiler_params=pltpu.CompilerParams(needs_layout_passes=False),
    scratch_types=[
        pltpu.VMEM((TPW * E,), jnp.float32),
        pltpu.VMEM((TPW * K,), jnp.int32),
        pltpu.VMEM((TPW * K,), jnp.float32),
        pltpu.VMEM((E,), jnp.float32),
    ],
)
def _route(logits_hbm, idx_hbm, prob_hbm, hist_hbm, lg_v, idx_v, prob_v, hist_v):
    wid = lax.axis_index("s") * 2 + lax.axis_index("c")
    base = wid * TPW
    pltpu.sync_copy(logits_hbm.at[pl.ds(base * E, TPW * E)], lg_v)

    hist_v[...] = jnp.zeros((E,), jnp.float32)
    lanes = lax.iota(jnp.int32, CHUNK)
    ones_f = jnp.ones((CHUNK,), jnp.float32)
    zero_i = jnp.zeros((CHUNK,), jnp.int32)

    for c in range(TPW // CHUNK):
        row_base = lanes * E + (c * CHUNK * E)
        m1 = plsc.load_gather(lg_v, [row_base])
        i1 = zero_i
        m2 = jnp.full((CHUNK,), -jnp.inf, jnp.float32)
        i2 = zero_i
        for e in range(1, E):
            e_i = jnp.full((CHUNK,), e, jnp.int32)
            v = plsc.load_gather(lg_v, [row_base + e])
            gt1 = v > m1
            gt2 = v > m2
            i2 = jnp.where(gt1, i1, jnp.where(gt2, e_i, i2))
            m2 = jnp.where(gt1, m1, jnp.where(gt2, v, m2))
            i1 = jnp.where(gt1, e_i, i1)
            m1 = jnp.where(gt1, v, m1)
        d = jnp.exp(m2 - m1)
        p1 = 1.0 / (1.0 + d)
        p2 = 1.0 - p1
        out_base = lanes * K + (c * CHUNK * K)
        plsc.store_scatter(idx_v, [out_base], i1)
        plsc.store_scatter(idx_v, [out_base + 1], i2)
        plsc.store_scatter(prob_v, [out_base], p1)
        plsc.store_scatter(prob_v, [out_base + 1], p2)
        plsc.addupdate_scatter(hist_v, [i1], ones_f)
        plsc.addupdate_scatter(hist_v, [i2], ones_f)

    pltpu.sync_copy(idx_v, idx_hbm.at[pl.ds(base * K, TPW * K)])
    pltpu.sync_copy(prob_v, prob_hbm.at[pl.ds(base * K, TPW * K)])
    pltpu.sync_copy(hist_v, hist_hbm.at[wid])


def kernel(x, W, b):
    logits = _logits(x, W, b.reshape(1, E))
    idx_flat, prob_flat, hist_parts = _route(logits.reshape(-1))
    return (
        idx_flat.reshape(N, K),
        prob_flat.reshape(N, K),
        hist_parts.sum(axis=0),
    )
